# Initial kernel scaffold; baseline (speedup 1.0000x reference)
import functools

import jax
import jax.numpy as jnp
from jax import lax
from jax.experimental import pallas as pl
from jax.experimental.pallas import tpu as pltpu
from jax.experimental.pallas import tpu_sc as plsc

N = 10000
E = 320000
D_IN = 128
D_HID = 16
D_OUT = 10

NC = 2
NS = 16
NW = NC * NS
CHUNK = 128
CH = 79
EPAD = NW * CH * CHUNK
TBL = 10240
SL = TBL // NS

_MESH = plsc.VectorSubcoreMesh(
    core_axis_name="c", subcore_axis_name="s", num_cores=NC, num_subcores=NS
)


def _mm1_body(x_ref, w_ref, o_ref):
    o_ref[...] = jnp.dot(x_ref[...], w_ref[...],
                         preferred_element_type=jnp.float32)


def _mid_body(p_ref, b_ref, w_ref, o_ref):
    h = jnp.maximum(p_ref[0] + p_ref[1] + b_ref[...], 0.0)
    o_ref[...] = jnp.dot(h, w_ref[...], preferred_element_type=jnp.float32)


def _fin_body(p_ref, b_ref, o_ref):
    o_ref[...] = p_ref[0, :N, :D_OUT] + p_ref[1, :N, :D_OUT] + b_ref[...]


@functools.partial(
    pl.kernel,
    out_type=jax.ShapeDtypeStruct((NC, TBL, D_HID), jnp.float32),
    mesh=_MESH,
    scratch_types=[
        pltpu.VMEM((CH, CHUNK), jnp.int32),
        pltpu.VMEM((CH, CHUNK), jnp.int32),
        pltpu.VMEM((CHUNK, D_HID), jnp.float32),
        pltpu.VMEM((SL, D_HID), jnp.float32),
        pltpu.VMEM_SHARED((TBL, D_HID), jnp.float32),
        pltpu.SemaphoreType.DMA,
    ],
)
def _edge_agg(table_h, src_h, dst_h, out_h, src_v, dst_v, rows_v, zb, acc,
              sem):
    cid = lax.axis_index("c")
    sid = lax.axis_index("s")
    wid = cid * NS + sid

    pltpu.sync_copy(src_h.at[wid], src_v)
    pltpu.sync_copy(dst_h.at[wid], dst_v)

    def zrow(i, carry):
        zb[i] = jnp.zeros((D_HID,), jnp.float32)
        return carry

    lax.fori_loop(0, SL, zrow, 0)
    pltpu.sync_copy(zb, acc.at[pl.ds(sid * SL, SL)])
    plsc.subcore_barrier()

    def chunk(j, carry):
        pltpu.async_copy(table_h.at[src_v.at[j]], rows_v, sem).wait()
        pltpu.sync_copy(rows_v, acc.at[dst_v.at[j]], add=True)
        return carry

    lax.fori_loop(0, CH, chunk, 0)
    plsc.subcore_barrier()
    pltpu.sync_copy(acc.at[pl.ds(sid * SL, SL)],
                    out_h.at[cid, pl.ds(sid * SL, SL)])


def kernel(x, edge_index, W1, b1, W2, b2):
    src = edge_index[0]
    dst = edge_index[1]
    pad = EPAD - E
    srcp = jnp.concatenate(
        [src, jnp.zeros((pad,), jnp.int32)]).reshape(NW, CH, CHUNK)
    dstp = jnp.concatenate(
        [dst, jnp.full((pad,), N, jnp.int32)]).reshape(NW, CH, CHUNK)

    xw = pl.pallas_call(
        _mm1_body,
        out_shape=jax.ShapeDtypeStruct((N, D_HID), jnp.float32),
    )(x, W1)

    p1 = _edge_agg(xw, srcp, dstp)

    w2p = jnp.pad(W2, ((0, 0), (0, D_HID - D_OUT)))
    hw = pl.pallas_call(
        _mid_body,
        out_shape=jax.ShapeDtypeStruct((TBL, D_HID), jnp.float32),
    )(p1, b1.reshape(1, D_HID), w2p)

    p2 = _edge_agg(hw, srcp, dstp)

    out = pl.pallas_call(
        _fin_body,
        out_shape=jax.ShapeDtypeStruct((N, D_OUT), jnp.float32),
    )(p2, b2.reshape(1, D_OUT))
    return out


# speedup vs baseline: 12.7064x; 12.7064x over previous
import functools

import jax
import jax.numpy as jnp
from jax import lax
from jax.experimental import pallas as pl
from jax.experimental.pallas import tpu as pltpu
from jax.experimental.pallas import tpu_sc as plsc

N = 10000
E = 320000
D_IN = 128
D_HID = 16
D_OUT = 10

NC = 2
NS = 16
NW = NC * NS
CHUNK = 128
CH = 79
EPAD = NW * CH * CHUNK
TBL = 10240
SL = TBL // NS

_MESH = plsc.VectorSubcoreMesh(
    core_axis_name="c", subcore_axis_name="s", num_cores=NC, num_subcores=NS
)


def _mm1_body(x_ref, w_ref, o_ref):
    o_ref[...] = jnp.dot(x_ref[...], w_ref[...],
                         preferred_element_type=jnp.float32)


def _mid_body(p_ref, b_ref, w_ref, o_ref):
    h = jnp.maximum(p_ref[0] + p_ref[1] + b_ref[...], 0.0)
    o_ref[...] = jnp.dot(h, w_ref[...], preferred_element_type=jnp.float32)


def _fin_body(p_ref, b_ref, o_ref):
    o_ref[...] = p_ref[0, :N, :D_OUT] + p_ref[1, :N, :D_OUT] + b_ref[...]


@functools.partial(
    pl.kernel,
    out_type=jax.ShapeDtypeStruct((NC, TBL, D_HID), jnp.float32),
    mesh=_MESH,
    compiler_params=pltpu.CompilerParams(use_tc_tiling_on_sc=False),
    scratch_types=[
        pltpu.VMEM((CH, CHUNK), jnp.int32),
        pltpu.VMEM((CH, CHUNK), jnp.int32),
        pltpu.VMEM((CHUNK, D_HID), jnp.float32),
        pltpu.VMEM((SL, D_HID), jnp.float32),
        pltpu.VMEM_SHARED((TBL, D_HID), jnp.float32),
        pltpu.SemaphoreType.DMA,
    ],
)
def _edge_agg(table_h, src_h, dst_h, out_h, src_v, dst_v, rows_v, zb, acc,
              sem):
    cid = lax.axis_index("c")
    sid = lax.axis_index("s")
    wid = cid * NS + sid

    pltpu.sync_copy(src_h.at[wid], src_v)
    pltpu.sync_copy(dst_h.at[wid], dst_v)

    def zrow(i, carry):
        zb[i] = jnp.zeros((D_HID,), jnp.float32)
        return carry

    lax.fori_loop(0, SL, zrow, 0)
    pltpu.sync_copy(zb, acc.at[pl.ds(sid * SL, SL)])
    plsc.subcore_barrier()

    def chunk(j, carry):
        pltpu.async_copy(table_h.at[src_v.at[j]], rows_v, sem).wait()
        pltpu.sync_copy(rows_v, acc.at[dst_v.at[j]], add=True)
        return carry

    lax.fori_loop(0, CH, chunk, 0)
    plsc.subcore_barrier()
    pltpu.sync_copy(acc.at[pl.ds(sid * SL, SL)],
                    out_h.at[cid, pl.ds(sid * SL, SL)])


def kernel(x, edge_index, W1, b1, W2, b2):
    src = edge_index[0]
    dst = edge_index[1]
    pad = EPAD - E
    srcp = jnp.concatenate(
        [src, jnp.zeros((pad,), jnp.int32)]).reshape(NW, CH, CHUNK)
    dstp = jnp.concatenate(
        [dst, jnp.full((pad,), N, jnp.int32)]).reshape(NW, CH, CHUNK)

    xw = pl.pallas_call(
        _mm1_body,
        out_shape=jax.ShapeDtypeStruct((N, D_HID), jnp.float32),
    )(x, W1)

    p1 = _edge_agg(xw, srcp, dstp)

    w2p = jnp.pad(W2, ((0, 0), (0, D_HID - D_OUT)))
    hw = pl.pallas_call(
        _mid_body,
        out_shape=jax.ShapeDtypeStruct((TBL, D_HID), jnp.float32),
    )(p1, b1.reshape(1, D_HID), w2p)

    p2 = _edge_agg(hw, srcp, dstp)

    out = pl.pallas_call(
        _fin_body,
        out_shape=jax.ShapeDtypeStruct((N, D_OUT), jnp.float32),
    )(p2, b2.reshape(1, D_OUT))
    return out


# speedup vs baseline: 16.7716x; 1.3199x over previous
import functools

import jax
import jax.numpy as jnp
from jax import lax
from jax.experimental import pallas as pl
from jax.experimental.pallas import tpu as pltpu
from jax.experimental.pallas import tpu_sc as plsc

N = 10000
E = 320000
D_IN = 128
D_HID = 16
D_OUT = 10

NC = 2
NS = 16
NW = NC * NS
CHUNK = 128
CH = 80
EPAD = NW * CH * CHUNK
TBL = 10240
SL = TBL // NS

_MESH = plsc.VectorSubcoreMesh(
    core_axis_name="c", subcore_axis_name="s", num_cores=NC, num_subcores=NS
)


def _mm1_body(x_ref, w_ref, o_ref):
    o_ref[...] = jnp.dot(x_ref[...], w_ref[...],
                         preferred_element_type=jnp.float32)


def _mid_body(p_ref, b_ref, w_ref, o_ref):
    h = jnp.maximum(p_ref[0] + p_ref[1] + b_ref[...], 0.0)
    o_ref[...] = jnp.dot(h, w_ref[...], preferred_element_type=jnp.float32)


def _fin_body(p_ref, b_ref, o_ref):
    o_ref[...] = p_ref[0, :N, :D_OUT] + p_ref[1, :N, :D_OUT] + b_ref[...]


@functools.partial(
    pl.kernel,
    out_type=jax.ShapeDtypeStruct((NC, TBL, D_HID), jnp.float32),
    mesh=_MESH,
    compiler_params=pltpu.CompilerParams(use_tc_tiling_on_sc=False),
    scratch_types=[
        pltpu.VMEM((CH, CHUNK), jnp.int32),
        pltpu.VMEM((CH, CHUNK), jnp.int32),
        pltpu.VMEM((CHUNK, D_HID), jnp.float32),
        pltpu.VMEM((CHUNK, D_HID), jnp.float32),
        pltpu.VMEM((64, D_HID), jnp.float32),
        pltpu.VMEM_SHARED((TBL, D_HID), jnp.float32),
        pltpu.SemaphoreType.DMA,
        pltpu.SemaphoreType.DMA,
        pltpu.SemaphoreType.DMA,
    ],
)
def _edge_agg(table_h, src_h, dst_h, out_h, src_v, dst_v, rows0, rows1, zb,
              acc, sem0, sem1, semi):
    cid = lax.axis_index("c")
    sid = lax.axis_index("s")
    wid = cid * NS + sid

    d_src = pltpu.async_copy(src_h.at[wid], src_v, semi)
    d_dst = pltpu.async_copy(dst_h.at[wid], dst_v, semi)

    def zrow(i, carry):
        zb[i] = jnp.zeros((D_HID,), jnp.float32)
        return carry

    lax.fori_loop(0, 64, zrow, 0)

    def zcopy(t, carry):
        pltpu.sync_copy(zb, acc.at[pl.ds(sid * SL + t * 64, 64)])
        return carry

    lax.fori_loop(0, SL // 64, zcopy, 0)
    d_src.wait()
    d_dst.wait()
    plsc.subcore_barrier()

    pltpu.async_copy(table_h.at[src_v.at[0]], rows0, sem0)

    def pair(g, carry):
        j0 = 2 * g
        j1 = j0 + 1
        pltpu.async_copy(table_h.at[src_v.at[j1]], rows1, sem1)
        pltpu.make_async_copy(table_h.at[src_v.at[j0]], rows0, sem0).wait()
        pltpu.sync_copy(rows0, acc.at[dst_v.at[j0]], add=True)
        pltpu.async_copy(table_h.at[src_v.at[j0 + 2]], rows0, sem0)
        pltpu.make_async_copy(table_h.at[src_v.at[j1]], rows1, sem1).wait()
        pltpu.sync_copy(rows1, acc.at[dst_v.at[j1]], add=True)
        return carry

    lax.fori_loop(0, CH // 2 - 1, pair, 0)
    jl0 = CH - 2
    jl1 = CH - 1
    pltpu.async_copy(table_h.at[src_v.at[jl1]], rows1, sem1)
    pltpu.make_async_copy(table_h.at[src_v.at[jl0]], rows0, sem0).wait()
    pltpu.sync_copy(rows0, acc.at[dst_v.at[jl0]], add=True)
    pltpu.make_async_copy(table_h.at[src_v.at[jl1]], rows1, sem1).wait()
    pltpu.sync_copy(rows1, acc.at[dst_v.at[jl1]], add=True)
    plsc.subcore_barrier()
    pltpu.sync_copy(acc.at[pl.ds(sid * SL, SL)],
                    out_h.at[cid, pl.ds(sid * SL, SL)])


def kernel(x, edge_index, W1, b1, W2, b2):
    src = edge_index[0]
    dst = edge_index[1]
    pad = EPAD - E
    srcp = jnp.concatenate(
        [src, jnp.zeros((pad,), jnp.int32)]).reshape(NW, CH, CHUNK)
    dstp = jnp.concatenate(
        [dst, jnp.full((pad,), N, jnp.int32)]).reshape(NW, CH, CHUNK)

    xw = pl.pallas_call(
        _mm1_body,
        out_shape=jax.ShapeDtypeStruct((N, D_HID), jnp.float32),
    )(x, W1)

    p1 = _edge_agg(xw, srcp, dstp)

    w2p = jnp.pad(W2, ((0, 0), (0, D_HID - D_OUT)))
    hw = pl.pallas_call(
        _mid_body,
        out_shape=jax.ShapeDtypeStruct((TBL, D_HID), jnp.float32),
    )(p1, b1.reshape(1, D_HID), w2p)

    p2 = _edge_agg(hw, srcp, dstp)

    out = pl.pallas_call(
        _fin_body,
        out_shape=jax.ShapeDtypeStruct((N, D_OUT), jnp.float32),
    )(p2, b2.reshape(1, D_OUT))
    return out
